# Initial kernel scaffold; baseline (speedup 1.0000x reference)
import jax
import jax.numpy as jnp
from jax.experimental import pallas as pl


def kernel(inp, points, next_pts, indices_, K, weight, bias, centers, w1, b1, w2, b2, w3, b3):
    raise NotImplementedError("write your pallas kernel here")


# speedup vs baseline: 2.3304x; 2.3304x over previous
import functools

import jax
import jax.numpy as jnp
from jax import lax
from jax.experimental import pallas as pl
from jax.experimental.pallas import tpu as pltpu
from jax.experimental.pallas import tpu_sc as plsc

_PTS_PAD = 16


def _sc_gather(feat_tbl, pts_tbl, idx_flat, chunk):
    E = idx_flat.shape[0]
    C = feat_tbl.shape[1]
    info = plsc.get_sparse_core_info()
    ncores, nsub = info.num_cores, info.num_subcores
    nw = ncores * nsub
    per_w = E // nw
    n_chunks = per_w // chunk

    mesh = plsc.VectorSubcoreMesh(core_axis_name="c", subcore_axis_name="s")

    @functools.partial(
        pl.kernel,
        mesh=mesh,
        compiler_params=pltpu.CompilerParams(use_tc_tiling_on_sc=False),
        out_type=[
            jax.ShapeDtypeStruct((E, C), jnp.float32),
            jax.ShapeDtypeStruct((E, _PTS_PAD), jnp.float32),
        ],
        scratch_types=[
            pltpu.VMEM((chunk,), jnp.int32),
            pltpu.VMEM((chunk, C), jnp.float32),
            pltpu.VMEM((chunk, _PTS_PAD), jnp.float32),
            pltpu.SemaphoreType.DMA,
            pltpu.SemaphoreType.DMA,
        ],
    )
    def gather_kernel(feat_hbm, pts_hbm, idx_hbm, feat_out, pts_out,
                      idx_v, feat_v, pts_v, sem_f, sem_p):
        wid = lax.axis_index("s") * ncores + lax.axis_index("c")
        base = wid * per_w

        def body(i, carry):
            off = base + i * chunk
            pltpu.sync_copy(idx_hbm.at[pl.ds(off, chunk)], idx_v)
            cp_f = pltpu.async_copy(feat_hbm.at[idx_v], feat_v, sem_f)
            cp_p = pltpu.async_copy(pts_hbm.at[idx_v], pts_v, sem_p)
            cp_f.wait()
            cp_p.wait()
            pltpu.sync_copy(feat_v, feat_out.at[pl.ds(off, chunk)])
            pltpu.sync_copy(pts_v, pts_out.at[pl.ds(off, chunk)])
            return carry

        lax.fori_loop(0, n_chunks, body, 0)

    return gather_kernel(feat_tbl, pts_tbl, idx_flat)


def _tc_body(featg_ref, ptsg_ref, nxt_ref, w1_ref, b1_ref, w2_ref, b2_ref,
             w3_ref, b3_ref, wn_ref, bias_ref, out_ref, *, kk, nc):
    p = nxt_ref.shape[0]
    c = featg_ref.shape[1]
    rel = ptsg_ref[...].reshape(p, kk, _PTS_PAD) - nxt_ref[...][:, None, :]
    rel = rel.reshape(p * kk, _PTS_PAD)
    h = jnp.dot(rel, w1_ref[...], preferred_element_type=jnp.float32)
    h = jnp.maximum(h + b1_ref[...], 0.0)
    h = jnp.dot(h, w2_ref[...], preferred_element_type=jnp.float32)
    h = jnp.maximum(h + b2_ref[...], 0.0)
    d = jnp.dot(h, w3_ref[...], preferred_element_type=jnp.float32)
    d = jnp.maximum(d + b3_ref[...], 0.0)
    d3 = d.reshape(p, kk, nc)
    feat3 = featg_ref[...].reshape(p, kk, c)
    acc = jnp.zeros((p, c), jnp.float32)
    for n in range(nc):
        s = jnp.sum(d3[:, :, n:n + 1] * feat3, axis=1)
        acc = acc + jnp.dot(s, wn_ref[n], preferred_element_type=jnp.float32)
    out_ref[...] = acc + bias_ref[...]


def kernel(inp, points, next_pts, indices_, K, weight, bias, centers,
           w1, b1, w2, b2, w3, b3):
    B, N, C_IN = inp.shape
    DIM = points.shape[2]
    NC = centers.shape[1]
    C_OUT = weight.shape[2]
    K = indices_.shape[2]
    E = B * N * K

    offs = (jnp.arange(B, dtype=jnp.int32) * N)[:, None, None]
    idx_flat = (indices_.astype(jnp.int32) + offs).reshape(E)
    feat_tbl = inp.reshape(B * N, C_IN)
    pts_tbl = jnp.pad(points.reshape(B * N, DIM), ((0, 0), (0, _PTS_PAD - DIM)))
    nxt_flat = jnp.pad(next_pts.reshape(B * N, DIM),
                       ((0, 0), (0, _PTS_PAD - DIM)))

    w1r = w1.reshape(2 * NC, DIM, NC)
    w1e = jnp.sum(w1r, axis=2).T
    w1p = jnp.pad(w1e, ((0, _PTS_PAD - DIM), (0, 0)))
    b1e = (b1 - jnp.sum(w1r * centers[None], axis=(1, 2))).reshape(1, 2 * NC)
    w2t = w2.T
    b2r = b2.reshape(1, NC)
    w3t = w3.T
    b3r = b3.reshape(1, NC)
    wn = jnp.transpose(weight, (1, 0, 2)) / K
    bias_r = bias.reshape(1, C_OUT)

    featg, ptsg = _sc_gather(feat_tbl, pts_tbl, idx_flat, chunk=1024)

    P = 512
    nb = (B * N) // P
    body = functools.partial(_tc_body, kk=K, nc=NC)
    out = pl.pallas_call(
        body,
        grid=(nb,),
        in_specs=[
            pl.BlockSpec((P * K, C_IN), lambda i: (i, 0)),
            pl.BlockSpec((P * K, _PTS_PAD), lambda i: (i, 0)),
            pl.BlockSpec((P, _PTS_PAD), lambda i: (i, 0)),
            pl.BlockSpec((_PTS_PAD, 2 * NC), lambda i: (0, 0)),
            pl.BlockSpec((1, 2 * NC), lambda i: (0, 0)),
            pl.BlockSpec((2 * NC, NC), lambda i: (0, 0)),
            pl.BlockSpec((1, NC), lambda i: (0, 0)),
            pl.BlockSpec((NC, NC), lambda i: (0, 0)),
            pl.BlockSpec((1, NC), lambda i: (0, 0)),
            pl.BlockSpec((NC, C_IN, C_OUT), lambda i: (0, 0, 0)),
            pl.BlockSpec((1, C_OUT), lambda i: (0, 0)),
        ],
        out_specs=pl.BlockSpec((P, C_OUT), lambda i: (i, 0)),
        out_shape=jax.ShapeDtypeStruct((B * N, C_OUT), jnp.float32),
    )(featg, ptsg, nxt_flat, w1p, b1e, w2t, b2r, w3t, b3r, wn, bias_r)

    return out.reshape(B, N, C_OUT)


# speedup vs baseline: 5.0551x; 2.1691x over previous
import functools

import jax
import jax.numpy as jnp
from jax import lax
from jax.experimental import pallas as pl
from jax.experimental.pallas import tpu as pltpu
from jax.experimental.pallas import tpu_sc as plsc

_PTS_PAD = 16


def _sc_gather(feat_tbl, pts_tbl, idx_flat, chunk):
    E = idx_flat.shape[0]
    C = feat_tbl.shape[1]
    info = plsc.get_sparse_core_info()
    ncores, nsub = info.num_cores, info.num_subcores
    nw = ncores * nsub
    per_w = E // nw
    n_chunks = per_w // chunk

    mesh = plsc.VectorSubcoreMesh(core_axis_name="c", subcore_axis_name="s")

    @functools.partial(
        pl.kernel,
        mesh=mesh,
        compiler_params=pltpu.CompilerParams(use_tc_tiling_on_sc=False),
        out_type=[
            jax.ShapeDtypeStruct((E, C), jnp.float32),
            jax.ShapeDtypeStruct((E, _PTS_PAD), jnp.float32),
        ],
        scratch_types=[
            pltpu.VMEM((chunk,), jnp.int32),
            pltpu.VMEM((chunk, C), jnp.float32),
            pltpu.VMEM((chunk, _PTS_PAD), jnp.float32),
            pltpu.SemaphoreType.DMA,
            pltpu.SemaphoreType.DMA,
        ],
    )
    def gather_kernel(feat_hbm, pts_hbm, idx_hbm, feat_out, pts_out,
                      idx_v, feat_v, pts_v, sem_f, sem_p):
        wid = lax.axis_index("s") * ncores + lax.axis_index("c")
        base = wid * per_w

        def body(i, carry):
            off = base + i * chunk
            pltpu.sync_copy(idx_hbm.at[pl.ds(off, chunk)], idx_v)
            cp_f = pltpu.async_copy(feat_hbm.at[idx_v], feat_v, sem_f)
            cp_p = pltpu.async_copy(pts_hbm.at[idx_v], pts_v, sem_p)
            cp_f.wait()
            cp_p.wait()
            pltpu.sync_copy(feat_v, feat_out.at[pl.ds(off, chunk)])
            pltpu.sync_copy(pts_v, pts_out.at[pl.ds(off, chunk)])
            return carry

        lax.fori_loop(0, n_chunks, body, 0)

    return gather_kernel(feat_tbl, pts_tbl, idx_flat)


def _tc_body(featg_ref, ptsg_ref, nxt_ref, w1_ref, b1_ref, w2_ref, b2_ref,
             w3_ref, b3_ref, wn_ref, bias_ref, out_ref, *, kk, nc):
    p = nxt_ref.shape[0]
    c = featg_ref.shape[1]
    rel = ptsg_ref[...].reshape(p, kk, _PTS_PAD) - nxt_ref[...][:, None, :]
    rel = rel.reshape(p * kk, _PTS_PAD)
    h = jnp.dot(rel, w1_ref[...], preferred_element_type=jnp.float32)
    h = jnp.maximum(h + b1_ref[...], 0.0)
    h = jnp.dot(h, w2_ref[...], preferred_element_type=jnp.float32)
    h = jnp.maximum(h + b2_ref[...], 0.0)
    d = jnp.dot(h, w3_ref[...], preferred_element_type=jnp.float32)
    d = jnp.maximum(d + b3_ref[...], 0.0)
    d3 = d.reshape(p, kk, nc)
    feat3 = featg_ref[...].reshape(p, kk, c)
    fpre = lax.dot_general(d3, feat3, (((1,), (1,)), ((0,), (0,))),
                           preferred_element_type=jnp.float32)
    out_ref[...] = jnp.dot(fpre.reshape(p, nc * c), wn_ref[...],
                           preferred_element_type=jnp.float32) + bias_ref[...]


def kernel(inp, points, next_pts, indices_, K, weight, bias, centers,
           w1, b1, w2, b2, w3, b3):
    B, N, C_IN = inp.shape
    DIM = points.shape[2]
    NC = centers.shape[1]
    C_OUT = weight.shape[2]
    K = indices_.shape[2]
    E = B * N * K

    offs = (jnp.arange(B, dtype=jnp.int32) * N)[:, None, None]
    idx_flat = (indices_.astype(jnp.int32) + offs).reshape(E)
    feat_tbl = inp.reshape(B * N, C_IN)
    pts_tbl = jnp.pad(points.reshape(B * N, DIM), ((0, 0), (0, _PTS_PAD - DIM)))
    nxt_flat = jnp.pad(next_pts.reshape(B * N, DIM),
                       ((0, 0), (0, _PTS_PAD - DIM)))

    w1r = w1.reshape(2 * NC, DIM, NC)
    w1e = jnp.sum(w1r, axis=2).T
    w1p = jnp.pad(w1e, ((0, _PTS_PAD - DIM), (0, 0)))
    b1e = (b1 - jnp.sum(w1r * centers[None], axis=(1, 2))).reshape(1, 2 * NC)
    w2t = w2.T
    b2r = b2.reshape(1, NC)
    w3t = w3.T
    b3r = b3.reshape(1, NC)
    wn = (jnp.transpose(weight, (1, 0, 2)) / K).reshape(NC * C_IN, C_OUT)
    bias_r = bias.reshape(1, C_OUT)

    featg, ptsg = _sc_gather(feat_tbl, pts_tbl, idx_flat, chunk=1024)

    P = 512
    nb = (B * N) // P
    body = functools.partial(_tc_body, kk=K, nc=NC)
    out = pl.pallas_call(
        body,
        grid=(nb,),
        in_specs=[
            pl.BlockSpec((P * K, C_IN), lambda i: (i, 0)),
            pl.BlockSpec((P * K, _PTS_PAD), lambda i: (i, 0)),
            pl.BlockSpec((P, _PTS_PAD), lambda i: (i, 0)),
            pl.BlockSpec((_PTS_PAD, 2 * NC), lambda i: (0, 0)),
            pl.BlockSpec((1, 2 * NC), lambda i: (0, 0)),
            pl.BlockSpec((2 * NC, NC), lambda i: (0, 0)),
            pl.BlockSpec((1, NC), lambda i: (0, 0)),
            pl.BlockSpec((NC, NC), lambda i: (0, 0)),
            pl.BlockSpec((1, NC), lambda i: (0, 0)),
            pl.BlockSpec((C_IN * NC, C_OUT), lambda i: (0, 0)),
            pl.BlockSpec((1, C_OUT), lambda i: (0, 0)),
        ],
        out_specs=pl.BlockSpec((P, C_OUT), lambda i: (i, 0)),
        out_shape=jax.ShapeDtypeStruct((B * N, C_OUT), jnp.float32),
    )(featg, ptsg, nxt_flat, w1p, b1e, w2t, b2r, w3t, b3r, wn, bias_r)

    return out.reshape(B, N, C_OUT)
